# Initial kernel scaffold; baseline (speedup 1.0000x reference)
import jax
import jax.numpy as jnp
from jax.experimental import pallas as pl


def kernel(h, e, edge_index, WA, bA, WB, bB, WC, bC, WD, bD, WE, bE, P):
    raise NotImplementedError("write your pallas kernel here")


# speedup vs baseline: 2.7018x; 2.7018x over previous
import functools

import jax
import jax.numpy as jnp
from jax import lax
from jax.experimental import pallas as pl
from jax.experimental.pallas import tpu as pltpu
from jax.experimental.pallas import tpu_sc as plsc

NC = 2
NS = 16


def _node_proj_body(h_ref, WB_ref, bB_ref, WD_ref, bD_ref, WE_ref, bE_ref,
                    p_ref, de_ref, bhp_ref):
    hb = h_ref[...]
    dn = (((1,), (1,)), ((), ()))
    Dh = lax.dot_general(hb, WD_ref[...], dn,
                         preferred_element_type=jnp.float32) + bD_ref[...]
    Eh = lax.dot_general(hb, WE_ref[...], dn,
                         preferred_element_type=jnp.float32) + bE_ref[...]
    Bh = lax.dot_general(hb, WB_ref[...], dn,
                         preferred_element_type=jnp.float32) + bB_ref[...]
    de_ref[0] = Dh
    de_ref[1] = Eh
    bhp_ref[...] = jnp.exp(p_ref[...] * jnp.log(jnp.abs(Bh)))


def _node_proj(h, WB, bB, WD, bD, WE, bE, p, R):
    N, D = h.shape
    grid = (N // R,)
    full = pl.BlockSpec((D, D), lambda i: (0, 0))
    vec = pl.BlockSpec((1, D), lambda i: (0, 0))
    return pl.pallas_call(
        _node_proj_body,
        grid=grid,
        in_specs=[
            pl.BlockSpec((R, D), lambda i: (i, 0)),
            full, vec, full, vec, full, vec, vec,
        ],
        out_specs=[
            pl.BlockSpec((2, R, D), lambda i: (0, i, 0)),
            pl.BlockSpec((R, D), lambda i: (i, 0)),
        ],
        out_shape=[
            jax.ShapeDtypeStruct((2, N, D), jnp.float32),
            jax.ShapeDtypeStruct((N, D), jnp.float32),
        ],
    )(h, WB, bB, WD, bD, WE, bE, p)


def _edge_body(e_ref, g1_ref, WC_ref, bC_ref, p_ref, enew_ref, sig_ref):
    eb = e_ref[...]
    dn = (((1,), (1,)), ((), ()))
    Ce = lax.dot_general(eb, WC_ref[...], dn,
                         preferred_element_type=jnp.float32) + bC_ref[...]
    e_out = Ce + g1_ref[...]
    sigma = jax.nn.sigmoid(e_out)
    sig_ref[...] = jnp.exp(p_ref[...] * jnp.log(sigma))
    enew_ref[...] = eb + jnp.maximum(e_out, 0.0)


def _edge_stage(e, g1, WC, bC, p, R):
    E, D = e.shape
    grid = (E // R,)
    full = pl.BlockSpec((D, D), lambda i: (0, 0))
    vec = pl.BlockSpec((1, D), lambda i: (0, 0))
    blk = pl.BlockSpec((R, D), lambda i: (i, 0))
    return pl.pallas_call(
        _edge_body,
        grid=grid,
        in_specs=[blk, blk, full, vec, vec],
        out_specs=[blk, blk],
        out_shape=[
            jax.ShapeDtypeStruct((E, D), jnp.float32),
            jax.ShapeDtypeStruct((E, D), jnp.float32),
        ],
    )(e, g1, WC, bC, p)


def _node_out_body(h_ref, WA_ref, bA_ref, sm_ref, ss_ref, p_ref, out_ref):
    hb = h_ref[...]
    dn = (((1,), (1,)), ((), ()))
    Ah = lax.dot_general(hb, WA_ref[...], dn,
                         preferred_element_type=jnp.float32) + bA_ref[...]
    ratio = sm_ref[...] / (ss_ref[...] + 1e-6)
    root = jnp.exp(jnp.log(ratio) / p_ref[...])
    out_ref[...] = hb + jnp.maximum(Ah + root, 0.0)


def _node_out(h, WA, bA, sum_m, sum_s, p, R):
    N, D = h.shape
    grid = (N // R,)
    full = pl.BlockSpec((D, D), lambda i: (0, 0))
    vec = pl.BlockSpec((1, D), lambda i: (0, 0))
    blk = pl.BlockSpec((R, D), lambda i: (i, 0))
    return pl.pallas_call(
        _node_out_body,
        grid=grid,
        in_specs=[blk, full, vec, blk, blk, vec],
        out_specs=blk,
        out_shape=jax.ShapeDtypeStruct((N, D), jnp.float32),
    )(h, WA, bA, sum_m, sum_s, p)


def _make_sc_gather(N2, E, D, B):
    EPW = E // (NC * NS)
    NCHUNK = EPW // B
    mesh = plsc.VectorSubcoreMesh(core_axis_name="c", subcore_axis_name="s")

    @functools.partial(
        pl.kernel,
        out_type=jax.ShapeDtypeStruct((E, D), jnp.float32),
        mesh=mesh,
        scratch_types=[
            pltpu.VMEM((B,), jnp.int32),
            pltpu.VMEM((B,), jnp.int32),
            pltpu.VMEM((B, D), jnp.float32),
            pltpu.VMEM((B, D), jnp.float32),
            pltpu.SemaphoreType.DMA,
            pltpu.SemaphoreType.DMA,
        ],
    )
    def sc_gather(table_hbm, src_hbm, dstn_hbm, out_hbm,
                  srcv, dstv, rows1, rows2, sem1, sem2):
        c = lax.axis_index("c")
        s = lax.axis_index("s")
        wid = s * NC + c

        def chunk(i, carry):
            base = wid * EPW + i * B
            pltpu.sync_copy(src_hbm.at[pl.ds(base, B)], srcv)
            pltpu.sync_copy(dstn_hbm.at[pl.ds(base, B)], dstv)
            cp1 = pltpu.async_copy(table_hbm.at[srcv], rows1, sem1)
            cp2 = pltpu.async_copy(table_hbm.at[dstv], rows2, sem2)
            cp1.wait()
            cp2.wait()

            def row(r, carry2):
                for j in range(D // 16):
                    sl = pl.ds(j * 16, 16)
                    rows1[r, sl] = rows1[r, sl] + rows2[r, sl]
                return carry2

            lax.fori_loop(0, B, row, 0)
            pltpu.sync_copy(rows1, out_hbm.at[pl.ds(base, B)])
            return carry

        lax.fori_loop(0, NCHUNK, chunk, 0)

    return sc_gather


def _make_sc_scatter(N, E, D, B):
    EPS = E // NS
    NCHUNK = EPS // B
    RPS = (N // NS) // 8 * 8
    TAIL = N - NS * RPS
    mesh = plsc.VectorSubcoreMesh(core_axis_name="c", subcore_axis_name="s")

    @functools.partial(
        pl.kernel,
        out_type=[
            jax.ShapeDtypeStruct((N, D), jnp.float32),
            jax.ShapeDtypeStruct((N, D), jnp.float32),
        ],
        mesh=mesh,
        scratch_types=[
            pltpu.VMEM((B,), jnp.int32),
            pltpu.VMEM((B,), jnp.int32),
            pltpu.VMEM((B, D), jnp.float32),
            pltpu.VMEM((B, D), jnp.float32),
            pltpu.VMEM_SHARED((N, D), jnp.float32),
            pltpu.SemaphoreType.DMA,
        ],
    )
    def sc_scatter(sig_hbm, bhp_hbm, src_hbm, dst_hbm, zero_hbm,
                   out_m, out_s, srcv, dstv, rows_sig, rows_bhp, acc, sem):
        c = lax.axis_index("c")
        s = lax.axis_index("s")

        pltpu.sync_copy(zero_hbm.at[pl.ds(s * RPS, RPS)],
                        acc.at[pl.ds(s * RPS, RPS)])

        @pl.when(s == 0)
        def _():
            pltpu.sync_copy(zero_hbm.at[pl.ds(NS * RPS, TAIL)],
                            acc.at[pl.ds(NS * RPS, TAIL)])

        plsc.subcore_barrier()

        def chunk_m(i, carry):
            base = s * EPS + i * B
            pltpu.sync_copy(src_hbm.at[pl.ds(base, B)], srcv)
            pltpu.sync_copy(dst_hbm.at[pl.ds(base, B)], dstv)
            cp = pltpu.async_copy(bhp_hbm.at[srcv], rows_bhp, sem)
            pltpu.sync_copy(sig_hbm.at[pl.ds(base, B)], rows_sig)
            cp.wait()

            def row(r, carry2):
                for j in range(D // 16):
                    sl = pl.ds(j * 16, 16)
                    rows_sig[r, sl] = rows_sig[r, sl] * rows_bhp[r, sl]
                return carry2

            lax.fori_loop(0, B, row, 0)
            pltpu.sync_copy(rows_sig, acc.at[dstv], add=True)
            return carry

        def chunk_s(i, carry):
            base = s * EPS + i * B
            pltpu.sync_copy(dst_hbm.at[pl.ds(base, B)], dstv)
            pltpu.sync_copy(sig_hbm.at[pl.ds(base, B)], rows_sig)
            pltpu.sync_copy(rows_sig, acc.at[dstv], add=True)
            return carry

        @pl.when(c == 0)
        def _():
            lax.fori_loop(0, NCHUNK, chunk_m, 0)

        @pl.when(c == 1)
        def _():
            lax.fori_loop(0, NCHUNK, chunk_s, 0)

        plsc.subcore_barrier()

        @pl.when(c == 0)
        def _():
            pltpu.sync_copy(acc.at[pl.ds(s * RPS, RPS)],
                            out_m.at[pl.ds(s * RPS, RPS)])

            @pl.when(s == 0)
            def _():
                pltpu.sync_copy(acc.at[pl.ds(NS * RPS, TAIL)],
                                out_m.at[pl.ds(NS * RPS, TAIL)])

        @pl.when(c == 1)
        def _():
            pltpu.sync_copy(acc.at[pl.ds(s * RPS, RPS)],
                            out_s.at[pl.ds(s * RPS, RPS)])

            @pl.when(s == 0)
            def _():
                pltpu.sync_copy(acc.at[pl.ds(NS * RPS, TAIL)],
                                out_s.at[pl.ds(NS * RPS, TAIL)])

    return sc_scatter


def kernel(h, e, edge_index, WA, bA, WB, bB, WC, bC, WD, bD, WE, bE, P):
    N, D = h.shape
    E = e.shape[0]

    src = edge_index[0]
    dst = edge_index[1]
    dstn = dst + N
    p = jnp.clip(P, 1.0, 100.0).reshape(1, D)
    zeros = jnp.zeros((N, D), jnp.float32)

    de, bhp = _node_proj(h, WB, bB.reshape(1, D), WD, bD.reshape(1, D),
                         WE, bE.reshape(1, D), p, R=1000)
    table = de.reshape(2 * N, D)

    g1 = _make_sc_gather(2 * N, E, D, B=400)(table, src, dstn)

    e_new, sig = _edge_stage(e, g1, WC, bC.reshape(1, D), p, R=512)

    sum_m, sum_s = _make_sc_scatter(N, E, D, B=160)(sig, bhp, src, dst, zeros)

    h_new = _node_out(h, WA, bA.reshape(1, D), sum_m, sum_s, p, R=1000)

    return (h_new, e_new)


# speedup vs baseline: 3.5844x; 1.3267x over previous
import functools

import jax
import jax.numpy as jnp
from jax import lax
from jax.experimental import pallas as pl
from jax.experimental.pallas import tpu as pltpu
from jax.experimental.pallas import tpu_sc as plsc

NC = 2
NS = 16


def _node_proj_body(h_ref, WB_ref, bB_ref, WD_ref, bD_ref, WE_ref, bE_ref,
                    p_ref, de_ref, bhp_ref):
    hb = h_ref[...]
    dn = (((1,), (1,)), ((), ()))
    Dh = lax.dot_general(hb, WD_ref[...], dn,
                         preferred_element_type=jnp.float32) + bD_ref[...]
    Eh = lax.dot_general(hb, WE_ref[...], dn,
                         preferred_element_type=jnp.float32) + bE_ref[...]
    Bh = lax.dot_general(hb, WB_ref[...], dn,
                         preferred_element_type=jnp.float32) + bB_ref[...]
    de_ref[0] = Dh
    de_ref[1] = Eh
    bhp_ref[...] = jnp.exp(p_ref[...] * jnp.log(jnp.abs(Bh)))


def _node_proj(h, WB, bB, WD, bD, WE, bE, p, R):
    N, D = h.shape
    grid = (N // R,)
    full = pl.BlockSpec((D, D), lambda i: (0, 0))
    vec = pl.BlockSpec((1, D), lambda i: (0, 0))
    return pl.pallas_call(
        _node_proj_body,
        grid=grid,
        in_specs=[
            pl.BlockSpec((R, D), lambda i: (i, 0)),
            full, vec, full, vec, full, vec, vec,
        ],
        out_specs=[
            pl.BlockSpec((2, R, D), lambda i: (0, i, 0)),
            pl.BlockSpec((R, D), lambda i: (i, 0)),
        ],
        out_shape=[
            jax.ShapeDtypeStruct((2, N, D), jnp.float32),
            jax.ShapeDtypeStruct((N, D), jnp.float32),
        ],
    )(h, WB, bB, WD, bD, WE, bE, p)


def _edge_body(e_ref, g1_ref, WC_ref, bC_ref, p_ref, enew_ref, sig_ref):
    eb = e_ref[...]
    dn = (((1,), (1,)), ((), ()))
    Ce = lax.dot_general(eb, WC_ref[...], dn,
                         preferred_element_type=jnp.float32) + bC_ref[...]
    e_out = Ce + g1_ref[...]
    sigma = jax.nn.sigmoid(e_out)
    sig_ref[...] = jnp.exp(p_ref[...] * jnp.log(sigma))
    enew_ref[...] = eb + jnp.maximum(e_out, 0.0)


def _edge_stage(e, g1, WC, bC, p, R):
    E, D = e.shape
    grid = (E // R,)
    full = pl.BlockSpec((D, D), lambda i: (0, 0))
    vec = pl.BlockSpec((1, D), lambda i: (0, 0))
    blk = pl.BlockSpec((R, D), lambda i: (i, 0))
    return pl.pallas_call(
        _edge_body,
        grid=grid,
        in_specs=[blk, blk, full, vec, vec],
        out_specs=[blk, blk],
        out_shape=[
            jax.ShapeDtypeStruct((E, D), jnp.float32),
            jax.ShapeDtypeStruct((E, D), jnp.float32),
        ],
    )(e, g1, WC, bC, p)


def _node_out_body(h_ref, WA_ref, bA_ref, sm_ref, ss_ref, p_ref, out_ref):
    hb = h_ref[...]
    dn = (((1,), (1,)), ((), ()))
    Ah = lax.dot_general(hb, WA_ref[...], dn,
                         preferred_element_type=jnp.float32) + bA_ref[...]
    ratio = sm_ref[...] / (ss_ref[...] + 1e-6)
    root = jnp.exp(jnp.log(ratio) / p_ref[...])
    out_ref[...] = hb + jnp.maximum(Ah + root, 0.0)


def _node_out(h, WA, bA, sum_m, sum_s, p, R):
    N, D = h.shape
    grid = (N // R,)
    full = pl.BlockSpec((D, D), lambda i: (0, 0))
    vec = pl.BlockSpec((1, D), lambda i: (0, 0))
    blk = pl.BlockSpec((R, D), lambda i: (i, 0))
    return pl.pallas_call(
        _node_out_body,
        grid=grid,
        in_specs=[blk, full, vec, blk, blk, vec],
        out_specs=blk,
        out_shape=jax.ShapeDtypeStruct((N, D), jnp.float32),
    )(h, WA, bA, sum_m, sum_s, p)


def _make_sc_gather(N2, E, D, B):
    EPW = E // (NC * NS)
    NCHUNK = EPW // B
    T = NCHUNK // 2
    mesh = plsc.VectorSubcoreMesh(core_axis_name="c", subcore_axis_name="s")

    @functools.partial(
        pl.kernel,
        out_type=jax.ShapeDtypeStruct((E, D), jnp.float32),
        mesh=mesh,
        scratch_types=[
            pltpu.VMEM((B,), jnp.int32),
            pltpu.VMEM((B,), jnp.int32),
            pltpu.VMEM((B,), jnp.int32),
            pltpu.VMEM((B,), jnp.int32),
            pltpu.VMEM((B, D), jnp.float32),
            pltpu.VMEM((B, D), jnp.float32),
            pltpu.VMEM((B, D), jnp.float32),
            pltpu.VMEM((B, D), jnp.float32),
        ] + [pltpu.SemaphoreType.DMA] * 10,
    )
    def sc_gather(table_hbm, src_hbm, dstn_hbm, out_hbm,
                  sv0, sv1, dv0, dv1, r1a, r1b, r2a, r2b,
                  ssi0, ssi1, sdi0, sdi1, sg10, sg11, sg20, sg21, wr0, wr1):
        c = lax.axis_index("c")
        s = lax.axis_index("s")
        wid = s * NC + c
        w0 = wid * EPW
        srcv = (sv0, sv1)
        dstv = (dv0, dv1)
        r1 = (r1a, r1b)
        r2 = (r2a, r2b)
        ssi = (ssi0, ssi1)
        sdi = (sdi0, sdi1)
        sg1 = (sg10, sg11)
        sg2 = (sg20, sg21)
        wr = (wr0, wr1)

        def add_rows(ra, rb):
            def row(r, carry):
                for j in range(D // 16):
                    sl = pl.ds(j * 16, 16)
                    ra[r, sl] = ra[r, sl] + rb[r, sl]
                return carry
            lax.fori_loop(0, B, row, 0)

        pltpu.sync_copy(src_hbm.at[pl.ds(w0, B)], srcv[0])
        pltpu.sync_copy(dstn_hbm.at[pl.ds(w0, B)], dstv[0])
        pltpu.async_copy(src_hbm.at[pl.ds(w0 + B, B)], srcv[1], ssi[1])
        pltpu.async_copy(dstn_hbm.at[pl.ds(w0 + B, B)], dstv[1], sdi[1])
        pltpu.async_copy(table_hbm.at[srcv[0]], r1[0], sg1[0])
        pltpu.async_copy(table_hbm.at[dstv[0]], r2[0], sg2[0])

        def outer(t, carry):
            for k in (0, 1):
                pi = k
                po = 1 - k
                g = 2 * t + k
                base = w0 + g * B

                if k == 0:
                    @pl.when(t > 0)
                    def _():
                        pltpu.make_async_copy(
                            r1[po], out_hbm.at[pl.ds(base - B, B)],
                            wr[po]).wait()
                else:
                    wrd[0].wait()

                if k == 0:
                    pltpu.make_async_copy(
                        table_hbm.at[srcv[pi]], r1[pi], sg1[pi]).wait()
                    pltpu.make_async_copy(
                        table_hbm.at[dstv[pi]], r2[pi], sg2[pi]).wait()
                else:
                    gd[0].wait()
                    gd[1].wait()

                @pl.when(g + 2 < NCHUNK)
                def _():
                    pltpu.async_copy(src_hbm.at[pl.ds(base + 2 * B, B)],
                                     srcv[pi], ssi[pi])
                    pltpu.async_copy(dstn_hbm.at[pl.ds(base + 2 * B, B)],
                                     dstv[pi], sdi[pi])

                @pl.when(g + 1 < NCHUNK)
                def _():
                    pltpu.make_async_copy(
                        src_hbm.at[pl.ds(base + B, B)], srcv[po],
                        ssi[po]).wait()
                    pltpu.make_async_copy(
                        dstn_hbm.at[pl.ds(base + B, B)], dstv[po],
                        sdi[po]).wait()
                    d1 = pltpu.async_copy(table_hbm.at[srcv[po]],
                                          r1[po], sg1[po])
                    d2 = pltpu.async_copy(table_hbm.at[dstv[po]],
                                          r2[po], sg2[po])

                if k == 0:
                    gd = (pltpu.make_async_copy(table_hbm.at[srcv[po]],
                                                r1[po], sg1[po]),
                          pltpu.make_async_copy(table_hbm.at[dstv[po]],
                                                r2[po], sg2[po]))

                add_rows(r1[pi], r2[pi])

                wd = pltpu.async_copy(r1[pi], out_hbm.at[pl.ds(base, B)],
                                      wr[pi])
                if k == 0:
                    wrd = (wd,)
            return carry

        lax.fori_loop(0, T, outer, 0)
        pltpu.make_async_copy(
            r1[1], out_hbm.at[pl.ds(w0 + (NCHUNK - 1) * B, B)], wr[1]).wait()

    return sc_gather


def _make_sc_scatter(N, E, D, B):
    EPS = E // NS
    NCHUNK = EPS // B
    RPS = (N // NS) // 8 * 8
    TAIL = N - NS * RPS
    mesh = plsc.VectorSubcoreMesh(core_axis_name="c", subcore_axis_name="s")

    T = NCHUNK // 2

    @functools.partial(
        pl.kernel,
        out_type=[
            jax.ShapeDtypeStruct((N, D), jnp.float32),
            jax.ShapeDtypeStruct((N, D), jnp.float32),
        ],
        mesh=mesh,
        scratch_types=[
            pltpu.VMEM((B,), jnp.int32),
            pltpu.VMEM((B,), jnp.int32),
            pltpu.VMEM((B,), jnp.int32),
            pltpu.VMEM((B,), jnp.int32),
            pltpu.VMEM((B, D), jnp.float32),
            pltpu.VMEM((B, D), jnp.float32),
            pltpu.VMEM((B, D), jnp.float32),
            pltpu.VMEM((B, D), jnp.float32),
            pltpu.VMEM_SHARED((N, D), jnp.float32),
        ] + [pltpu.SemaphoreType.DMA] * 10,
    )
    def sc_scatter(sig_hbm, bhp_hbm, src_hbm, dst_hbm, zero_hbm,
                   out_m, out_s, sv0, sv1, dv0, dv1, sa, sb, ba, bb, acc,
                   ssi0, ssi1, sdi0, sdi1, sg0, sg1, ss0, ss1, sc0, sc1):
        c = lax.axis_index("c")
        s = lax.axis_index("s")
        w0 = s * EPS
        srcv = (sv0, sv1)
        dstv = (dv0, dv1)
        rs = (sa, sb)
        rb = (ba, bb)
        ssi = (ssi0, ssi1)
        sdi = (sdi0, sdi1)
        sg = (sg0, sg1)
        ss = (ss0, ss1)
        sc = (sc0, sc1)

        pltpu.sync_copy(zero_hbm.at[pl.ds(s * RPS, RPS)],
                        acc.at[pl.ds(s * RPS, RPS)])

        @pl.when(s == 0)
        def _():
            pltpu.sync_copy(zero_hbm.at[pl.ds(NS * RPS, TAIL)],
                            acc.at[pl.ds(NS * RPS, TAIL)])

        plsc.subcore_barrier()

        def mul_rows(ra, rbb):
            def row(r, carry):
                for j in range(D // 16):
                    sl = pl.ds(j * 16, 16)
                    ra[r, sl] = ra[r, sl] * rbb[r, sl]
                return carry
            lax.fori_loop(0, B, row, 0)

        @pl.when(c == 0)
        def _():
            pltpu.sync_copy(src_hbm.at[pl.ds(w0, B)], srcv[0])
            pltpu.sync_copy(dst_hbm.at[pl.ds(w0, B)], dstv[0])
            pltpu.async_copy(src_hbm.at[pl.ds(w0 + B, B)], srcv[1], ssi[1])
            pltpu.async_copy(bhp_hbm.at[srcv[0]], rb[0], sg[0])
            pltpu.async_copy(sig_hbm.at[pl.ds(w0, B)], rs[0], ss[0])

            def outer(t, carry):
                for k in (0, 1):
                    pi = k
                    po = 1 - k
                    g = 2 * t + k
                    base = w0 + g * B

                    if k == 0:
                        @pl.when(t > 0)
                        def _():
                            pltpu.make_async_copy(
                                rs[po], acc.at[dstv[po]], sc[po]).wait()
                    else:
                        scd[0].wait()

                    @pl.when(g + 1 < NCHUNK)
                    def _():
                        pltpu.async_copy(dst_hbm.at[pl.ds(base + B, B)],
                                         dstv[po], sdi[po])

                    if k == 0:
                        pltpu.make_async_copy(
                            bhp_hbm.at[srcv[pi]], rb[pi], sg[pi]).wait()
                        pltpu.make_async_copy(
                            sig_hbm.at[pl.ds(base, B)], rs[pi],
                            ss[pi]).wait()
                    else:
                        gd[0].wait()
                        gd[1].wait()

                    @pl.when(g + 2 < NCHUNK)
                    def _():
                        pltpu.async_copy(src_hbm.at[pl.ds(base + 2 * B, B)],
                                         srcv[pi], ssi[pi])

                    @pl.when(g + 1 < NCHUNK)
                    def _():
                        pltpu.make_async_copy(
                            src_hbm.at[pl.ds(base + B, B)], srcv[po],
                            ssi[po]).wait()
                        pltpu.make_async_copy(
                            dst_hbm.at[pl.ds(base + B, B)], dstv[po],
                            sdi[po]).wait()
                        pltpu.async_copy(bhp_hbm.at[srcv[po]],
                                         rb[po], sg[po])
                        pltpu.async_copy(sig_hbm.at[pl.ds(base + B, B)],
                                         rs[po], ss[po])

                    if k == 0:
                        gd = (pltpu.make_async_copy(bhp_hbm.at[srcv[po]],
                                                    rb[po], sg[po]),
                              pltpu.make_async_copy(
                                  sig_hbm.at[pl.ds(base + B, B)], rs[po],
                                  ss[po]))

                    mul_rows(rs[pi], rb[pi])

                    sd = pltpu.async_copy(rs[pi], acc.at[dstv[pi]],
                                          sc[pi], add=True)
                    if k == 0:
                        scd = (sd,)
                return carry

            lax.fori_loop(0, T, outer, 0)
            pltpu.make_async_copy(rs[1], acc.at[dstv[1]], sc[1]).wait()

        @pl.when(c == 1)
        def _():
            pltpu.async_copy(sig_hbm.at[pl.ds(w0, B)], rs[0], ss[0])
            pltpu.async_copy(dst_hbm.at[pl.ds(w0, B)], dstv[0], sdi[0])

            def outer(t, carry):
                for k in (0, 1):
                    pi = k
                    po = 1 - k
                    g = 2 * t + k
                    base = w0 + g * B

                    if k == 0:
                        @pl.when(t > 0)
                        def _():
                            pltpu.make_async_copy(
                                rs[po], acc.at[dstv[po]], sc[po]).wait()
                    else:
                        scd[0].wait()

                    @pl.when(g + 1 < NCHUNK)
                    def _():
                        pltpu.async_copy(sig_hbm.at[pl.ds(base + B, B)],
                                         rs[po], ss[po])
                        pltpu.async_copy(dst_hbm.at[pl.ds(base + B, B)],
                                         dstv[po], sdi[po])

                    pltpu.make_async_copy(
                        sig_hbm.at[pl.ds(base, B)], rs[pi], ss[pi]).wait()
                    pltpu.make_async_copy(
                        dst_hbm.at[pl.ds(base, B)], dstv[pi],
                        sdi[pi]).wait()

                    sd = pltpu.async_copy(rs[pi], acc.at[dstv[pi]],
                                          sc[pi], add=True)
                    if k == 0:
                        scd = (sd,)
                return carry

            lax.fori_loop(0, T, outer, 0)
            pltpu.make_async_copy(rs[1], acc.at[dstv[1]], sc[1]).wait()

        plsc.subcore_barrier()

        @pl.when(c == 0)
        def _():
            pltpu.sync_copy(acc.at[pl.ds(s * RPS, RPS)],
                            out_m.at[pl.ds(s * RPS, RPS)])

            @pl.when(s == 0)
            def _():
                pltpu.sync_copy(acc.at[pl.ds(NS * RPS, TAIL)],
                                out_m.at[pl.ds(NS * RPS, TAIL)])

        @pl.when(c == 1)
        def _():
            pltpu.sync_copy(acc.at[pl.ds(s * RPS, RPS)],
                            out_s.at[pl.ds(s * RPS, RPS)])

            @pl.when(s == 0)
            def _():
                pltpu.sync_copy(acc.at[pl.ds(NS * RPS, TAIL)],
                                out_s.at[pl.ds(NS * RPS, TAIL)])

    return sc_scatter


def kernel(h, e, edge_index, WA, bA, WB, bB, WC, bC, WD, bD, WE, bE, P):
    N, D = h.shape
    E = e.shape[0]

    src = edge_index[0]
    dst = edge_index[1]
    dstn = dst + N
    p = jnp.clip(P, 1.0, 100.0).reshape(1, D)
    zeros = jnp.zeros((N, D), jnp.float32)

    de, bhp = _node_proj(h, WB, bB.reshape(1, D), WD, bD.reshape(1, D),
                         WE, bE.reshape(1, D), p, R=1000)
    table = de.reshape(2 * N, D)

    g1 = _make_sc_gather(2 * N, E, D, B=200)(table, src, dstn)

    e_new, sig = _edge_stage(e, g1, WC, bC.reshape(1, D), p, R=512)

    sum_m, sum_s = _make_sc_scatter(N, E, D, B=80)(sig, bhp, src, dst, zeros)

    h_new = _node_out(h, WA, bA.reshape(1, D), sum_m, sum_s, p, R=1000)

    return (h_new, e_new)


# speedup vs baseline: 4.9666x; 1.3856x over previous
import functools

import jax
import jax.numpy as jnp
from jax import lax
from jax.experimental import pallas as pl
from jax.experimental.pallas import tpu as pltpu
from jax.experimental.pallas import tpu_sc as plsc

NC = 2
NS = 16


def _node_proj_body(h_ref, WB_ref, bB_ref, WD_ref, bD_ref, WE_ref, bE_ref,
                    p_ref, de_ref, bhp_ref):
    hb = h_ref[...]
    dn = (((1,), (1,)), ((), ()))
    Dh = lax.dot_general(hb, WD_ref[...], dn,
                         preferred_element_type=jnp.float32) + bD_ref[...]
    Eh = lax.dot_general(hb, WE_ref[...], dn,
                         preferred_element_type=jnp.float32) + bE_ref[...]
    Bh = lax.dot_general(hb, WB_ref[...], dn,
                         preferred_element_type=jnp.float32) + bB_ref[...]
    de_ref[0] = Dh
    de_ref[1] = Eh
    bhp_ref[...] = jnp.exp(p_ref[...] * jnp.log(jnp.abs(Bh)))


def _node_proj(h, WB, bB, WD, bD, WE, bE, p, R):
    N, D = h.shape
    grid = (N // R,)
    full = pl.BlockSpec((D, D), lambda i: (0, 0))
    vec = pl.BlockSpec((1, D), lambda i: (0, 0))
    return pl.pallas_call(
        _node_proj_body,
        grid=grid,
        in_specs=[
            pl.BlockSpec((R, D), lambda i: (i, 0)),
            full, vec, full, vec, full, vec, vec,
        ],
        out_specs=[
            pl.BlockSpec((2, R, D), lambda i: (0, i, 0)),
            pl.BlockSpec((R, D), lambda i: (i, 0)),
        ],
        out_shape=[
            jax.ShapeDtypeStruct((2, N, D), jnp.float32),
            jax.ShapeDtypeStruct((N, D), jnp.float32),
        ],
    )(h, WB, bB, WD, bD, WE, bE, p)


def _edge_body(e_ref, g1_ref, WC_ref, bC_ref, p_ref, enew_ref, sig_ref):
    eb = e_ref[...]
    dn = (((1,), (1,)), ((), ()))
    Ce = lax.dot_general(eb, WC_ref[...], dn,
                         preferred_element_type=jnp.float32) + bC_ref[...]
    e_out = Ce + g1_ref[...]
    sigma = jax.nn.sigmoid(e_out)
    sig_ref[...] = jnp.exp(p_ref[...] * jnp.log(sigma))
    enew_ref[...] = eb + jnp.maximum(e_out, 0.0)


def _edge_body_alias(enew_in_ref, e_ref, g1_ref, WC_ref, bC_ref, p_ref,
                     enew_ref, sig_ref):
    _edge_body(e_ref, g1_ref, WC_ref, bC_ref, p_ref, enew_ref, sig_ref)


def _edge_stage(e, g1_k, WC, bC, p, R, koff, enew_in):
    E, D = e.shape
    Es = g1_k.shape[0]
    grid = (Es // R,)
    full = pl.BlockSpec((D, D), lambda i: (0, 0))
    vec = pl.BlockSpec((1, D), lambda i: (0, 0))
    blk = pl.BlockSpec((R, D), lambda i: (i, 0))
    eblk = pl.BlockSpec((R, D), lambda i: (koff + i, 0))
    out_shape = [
        jax.ShapeDtypeStruct((E, D), jnp.float32),
        jax.ShapeDtypeStruct((Es, D), jnp.float32),
    ]
    if enew_in is None:
        return pl.pallas_call(
            _edge_body,
            grid=grid,
            in_specs=[eblk, blk, full, vec, vec],
            out_specs=[eblk, blk],
            out_shape=out_shape,
        )(e, g1_k, WC, bC, p)
    return pl.pallas_call(
        _edge_body_alias,
        grid=grid,
        in_specs=[pl.BlockSpec(memory_space=pltpu.HBM),
                  eblk, blk, full, vec, vec],
        out_specs=[eblk, blk],
        out_shape=out_shape,
        input_output_aliases={0: 0},
    )(enew_in, e, g1_k, WC, bC, p)


def _make_node_out_body(K):
    def body(*refs):
        h_ref, WA_ref, bA_ref = refs[0], refs[1], refs[2]
        sm_refs = refs[3:3 + K]
        ss_refs = refs[3 + K:3 + 2 * K]
        p_ref = refs[3 + 2 * K]
        out_ref = refs[4 + 2 * K]
        hb = h_ref[...]
        dn = (((1,), (1,)), ((), ()))
        Ah = lax.dot_general(hb, WA_ref[...], dn,
                             preferred_element_type=jnp.float32) + bA_ref[...]
        sm = sm_refs[0][...]
        for r in sm_refs[1:]:
            sm = sm + r[...]
        ss = ss_refs[0][...]
        for r in ss_refs[1:]:
            ss = ss + r[...]
        ratio = sm / (ss + 1e-6)
        root = jnp.exp(jnp.log(ratio) / p_ref[...])
        out_ref[...] = hb + jnp.maximum(Ah + root, 0.0)
    return body


def _node_out(h, WA, bA, sum_ms, sum_ss, p, R):
    N, D = h.shape
    K = len(sum_ms)
    grid = (N // R,)
    full = pl.BlockSpec((D, D), lambda i: (0, 0))
    vec = pl.BlockSpec((1, D), lambda i: (0, 0))
    blk = pl.BlockSpec((R, D), lambda i: (i, 0))
    return pl.pallas_call(
        _make_node_out_body(K),
        grid=grid,
        in_specs=[blk, full, vec] + [blk] * (2 * K) + [vec],
        out_specs=blk,
        out_shape=jax.ShapeDtypeStruct((N, D), jnp.float32),
    )(h, WA, bA, *sum_ms, *sum_ss, p)


def _make_sc_gather(N2, E, D, B, goff):
    EPW = E // (NC * NS)
    NCHUNK = EPW // B
    T = NCHUNK // 2
    mesh = plsc.VectorSubcoreMesh(core_axis_name="c", subcore_axis_name="s")

    @functools.partial(
        pl.kernel,
        out_type=jax.ShapeDtypeStruct((E, D), jnp.float32),
        mesh=mesh,
        scratch_types=[
            pltpu.VMEM((B,), jnp.int32),
            pltpu.VMEM((B,), jnp.int32),
            pltpu.VMEM((B,), jnp.int32),
            pltpu.VMEM((B,), jnp.int32),
            pltpu.VMEM((B, D), jnp.float32),
            pltpu.VMEM((B, D), jnp.float32),
            pltpu.VMEM((B, D), jnp.float32),
            pltpu.VMEM((B, D), jnp.float32),
        ] + [pltpu.SemaphoreType.DMA] * 10,
    )
    def sc_gather(table_hbm, src_hbm, dstn_hbm, out_hbm,
                  sv0, sv1, dv0, dv1, r1a, r1b, r2a, r2b,
                  ssi0, ssi1, sdi0, sdi1, sg10, sg11, sg20, sg21, wr0, wr1):
        c = lax.axis_index("c")
        s = lax.axis_index("s")
        wid = s * NC + c
        w0 = goff + wid * EPW
        o0 = wid * EPW
        srcv = (sv0, sv1)
        dstv = (dv0, dv1)
        r1 = (r1a, r1b)
        r2 = (r2a, r2b)
        ssi = (ssi0, ssi1)
        sdi = (sdi0, sdi1)
        sg1 = (sg10, sg11)
        sg2 = (sg20, sg21)
        wr = (wr0, wr1)

        def add_rows(ra, rb):
            def row(r, carry):
                for j in range(D // 16):
                    sl = pl.ds(j * 16, 16)
                    ra[r, sl] = ra[r, sl] + rb[r, sl]
                return carry
            lax.fori_loop(0, B, row, 0)

        pltpu.sync_copy(src_hbm.at[pl.ds(w0, B)], srcv[0])
        pltpu.sync_copy(dstn_hbm.at[pl.ds(w0, B)], dstv[0])
        pltpu.async_copy(src_hbm.at[pl.ds(w0 + B, B)], srcv[1], ssi[1])
        pltpu.async_copy(dstn_hbm.at[pl.ds(w0 + B, B)], dstv[1], sdi[1])
        pltpu.async_copy(table_hbm.at[srcv[0]], r1[0], sg1[0])
        pltpu.async_copy(table_hbm.at[dstv[0]], r2[0], sg2[0])

        def outer(t, carry):
            for k in (0, 1):
                pi = k
                po = 1 - k
                g = 2 * t + k
                base = w0 + g * B
                obase = o0 + g * B

                if k == 0:
                    @pl.when(t > 0)
                    def _():
                        pltpu.make_async_copy(
                            r1[po], out_hbm.at[pl.ds(obase - B, B)],
                            wr[po]).wait()
                else:
                    wrd[0].wait()

                if k == 0:
                    pltpu.make_async_copy(
                        table_hbm.at[srcv[pi]], r1[pi], sg1[pi]).wait()
                    pltpu.make_async_copy(
                        table_hbm.at[dstv[pi]], r2[pi], sg2[pi]).wait()
                else:
                    gd[0].wait()
                    gd[1].wait()

                @pl.when(g + 2 < NCHUNK)
                def _():
                    pltpu.async_copy(src_hbm.at[pl.ds(base + 2 * B, B)],
                                     srcv[pi], ssi[pi])
                    pltpu.async_copy(dstn_hbm.at[pl.ds(base + 2 * B, B)],
                                     dstv[pi], sdi[pi])

                @pl.when(g + 1 < NCHUNK)
                def _():
                    pltpu.make_async_copy(
                        src_hbm.at[pl.ds(base + B, B)], srcv[po],
                        ssi[po]).wait()
                    pltpu.make_async_copy(
                        dstn_hbm.at[pl.ds(base + B, B)], dstv[po],
                        sdi[po]).wait()
                    d1 = pltpu.async_copy(table_hbm.at[srcv[po]],
                                          r1[po], sg1[po])
                    d2 = pltpu.async_copy(table_hbm.at[dstv[po]],
                                          r2[po], sg2[po])

                if k == 0:
                    gd = (pltpu.make_async_copy(table_hbm.at[srcv[po]],
                                                r1[po], sg1[po]),
                          pltpu.make_async_copy(table_hbm.at[dstv[po]],
                                                r2[po], sg2[po]))

                add_rows(r1[pi], r2[pi])

                wd = pltpu.async_copy(r1[pi], out_hbm.at[pl.ds(obase, B)],
                                      wr[pi])
                if k == 0:
                    wrd = (wd,)
            return carry

        lax.fori_loop(0, T, outer, 0)
        pltpu.make_async_copy(
            r1[1], out_hbm.at[pl.ds(o0 + (NCHUNK - 1) * B, B)], wr[1]).wait()

    return sc_gather


def _make_sc_scatter(N, E, D, B, goff):
    EPS = E // NS
    NCHUNK = EPS // B
    RPS = (N // NS) // 8 * 8
    TAIL = N - NS * RPS
    mesh = plsc.VectorSubcoreMesh(core_axis_name="c", subcore_axis_name="s")

    T = NCHUNK // 2

    @functools.partial(
        pl.kernel,
        out_type=[
            jax.ShapeDtypeStruct((N, D), jnp.float32),
            jax.ShapeDtypeStruct((N, D), jnp.float32),
        ],
        mesh=mesh,
        scratch_types=[
            pltpu.VMEM((B,), jnp.int32),
            pltpu.VMEM((B,), jnp.int32),
            pltpu.VMEM((B,), jnp.int32),
            pltpu.VMEM((B,), jnp.int32),
            pltpu.VMEM((B, D), jnp.float32),
            pltpu.VMEM((B, D), jnp.float32),
            pltpu.VMEM((B, D), jnp.float32),
            pltpu.VMEM((B, D), jnp.float32),
            pltpu.VMEM_SHARED((N, D), jnp.float32),
        ] + [pltpu.SemaphoreType.DMA] * 10,
    )
    def sc_scatter(sig_hbm, bhp_hbm, src_hbm, dst_hbm, zero_hbm,
                   out_m, out_s, sv0, sv1, dv0, dv1, sa, sb, ba, bb, acc,
                   ssi0, ssi1, sdi0, sdi1, sg0, sg1, ss0, ss1, sc0, sc1):
        c = lax.axis_index("c")
        s = lax.axis_index("s")
        w0 = goff + s * EPS
        l0 = s * EPS
        srcv = (sv0, sv1)
        dstv = (dv0, dv1)
        rs = (sa, sb)
        rb = (ba, bb)
        ssi = (ssi0, ssi1)
        sdi = (sdi0, sdi1)
        sg = (sg0, sg1)
        ss = (ss0, ss1)
        sc = (sc0, sc1)

        pltpu.sync_copy(zero_hbm.at[pl.ds(s * RPS, RPS)],
                        acc.at[pl.ds(s * RPS, RPS)])

        @pl.when(s == 0)
        def _():
            pltpu.sync_copy(zero_hbm.at[pl.ds(NS * RPS, TAIL)],
                            acc.at[pl.ds(NS * RPS, TAIL)])

        plsc.subcore_barrier()

        def mul_rows(ra, rbb):
            def row(r, carry):
                for j in range(D // 16):
                    sl = pl.ds(j * 16, 16)
                    ra[r, sl] = ra[r, sl] * rbb[r, sl]
                return carry
            lax.fori_loop(0, B, row, 0)

        @pl.when(c == 0)
        def _():
            pltpu.sync_copy(src_hbm.at[pl.ds(w0, B)], srcv[0])
            pltpu.sync_copy(dst_hbm.at[pl.ds(w0, B)], dstv[0])
            pltpu.async_copy(src_hbm.at[pl.ds(w0 + B, B)], srcv[1], ssi[1])
            pltpu.async_copy(bhp_hbm.at[srcv[0]], rb[0], sg[0])
            pltpu.async_copy(sig_hbm.at[pl.ds(l0, B)], rs[0], ss[0])

            def outer(t, carry):
                for k in (0, 1):
                    pi = k
                    po = 1 - k
                    g = 2 * t + k
                    base = w0 + g * B
                    lbase = l0 + g * B

                    if k == 0:
                        @pl.when(t > 0)
                        def _():
                            pltpu.make_async_copy(
                                rs[po], acc.at[dstv[po]], sc[po]).wait()
                    else:
                        scd[0].wait()

                    @pl.when(g + 1 < NCHUNK)
                    def _():
                        pltpu.async_copy(dst_hbm.at[pl.ds(base + B, B)],
                                         dstv[po], sdi[po])

                    if k == 0:
                        pltpu.make_async_copy(
                            bhp_hbm.at[srcv[pi]], rb[pi], sg[pi]).wait()
                        pltpu.make_async_copy(
                            sig_hbm.at[pl.ds(lbase, B)], rs[pi],
                            ss[pi]).wait()
                    else:
                        gd[0].wait()
                        gd[1].wait()

                    @pl.when(g + 2 < NCHUNK)
                    def _():
                        pltpu.async_copy(src_hbm.at[pl.ds(base + 2 * B, B)],
                                         srcv[pi], ssi[pi])

                    @pl.when(g + 1 < NCHUNK)
                    def _():
                        pltpu.make_async_copy(
                            src_hbm.at[pl.ds(base + B, B)], srcv[po],
                            ssi[po]).wait()
                        pltpu.make_async_copy(
                            dst_hbm.at[pl.ds(base + B, B)], dstv[po],
                            sdi[po]).wait()
                        pltpu.async_copy(bhp_hbm.at[srcv[po]],
                                         rb[po], sg[po])
                        pltpu.async_copy(sig_hbm.at[pl.ds(lbase + B, B)],
                                         rs[po], ss[po])

                    if k == 0:
                        gd = (pltpu.make_async_copy(bhp_hbm.at[srcv[po]],
                                                    rb[po], sg[po]),
                              pltpu.make_async_copy(
                                  sig_hbm.at[pl.ds(lbase + B, B)], rs[po],
                                  ss[po]))

                    mul_rows(rs[pi], rb[pi])

                    sd = pltpu.async_copy(rs[pi], acc.at[dstv[pi]],
                                          sc[pi], add=True)
                    if k == 0:
                        scd = (sd,)
                return carry

            lax.fori_loop(0, T, outer, 0)
            pltpu.make_async_copy(rs[1], acc.at[dstv[1]], sc[1]).wait()

        @pl.when(c == 1)
        def _():
            pltpu.async_copy(sig_hbm.at[pl.ds(l0, B)], rs[0], ss[0])
            pltpu.async_copy(dst_hbm.at[pl.ds(w0, B)], dstv[0], sdi[0])

            def outer(t, carry):
                for k in (0, 1):
                    pi = k
                    po = 1 - k
                    g = 2 * t + k
                    base = w0 + g * B
                    lbase = l0 + g * B

                    if k == 0:
                        @pl.when(t > 0)
                        def _():
                            pltpu.make_async_copy(
                                rs[po], acc.at[dstv[po]], sc[po]).wait()
                    else:
                        scd[0].wait()

                    @pl.when(g + 1 < NCHUNK)
                    def _():
                        pltpu.async_copy(sig_hbm.at[pl.ds(lbase + B, B)],
                                         rs[po], ss[po])
                        pltpu.async_copy(dst_hbm.at[pl.ds(base + B, B)],
                                         dstv[po], sdi[po])

                    pltpu.make_async_copy(
                        sig_hbm.at[pl.ds(lbase, B)], rs[pi], ss[pi]).wait()
                    pltpu.make_async_copy(
                        dst_hbm.at[pl.ds(base, B)], dstv[pi],
                        sdi[pi]).wait()

                    sd = pltpu.async_copy(rs[pi], acc.at[dstv[pi]],
                                          sc[pi], add=True)
                    if k == 0:
                        scd = (sd,)
                return carry

            lax.fori_loop(0, T, outer, 0)
            pltpu.make_async_copy(rs[1], acc.at[dstv[1]], sc[1]).wait()

        plsc.subcore_barrier()

        @pl.when(c == 0)
        def _():
            pltpu.sync_copy(acc.at[pl.ds(s * RPS, RPS)],
                            out_m.at[pl.ds(s * RPS, RPS)])

            @pl.when(s == 0)
            def _():
                pltpu.sync_copy(acc.at[pl.ds(NS * RPS, TAIL)],
                                out_m.at[pl.ds(NS * RPS, TAIL)])

        @pl.when(c == 1)
        def _():
            pltpu.sync_copy(acc.at[pl.ds(s * RPS, RPS)],
                            out_s.at[pl.ds(s * RPS, RPS)])

            @pl.when(s == 0)
            def _():
                pltpu.sync_copy(acc.at[pl.ds(NS * RPS, TAIL)],
                                out_s.at[pl.ds(NS * RPS, TAIL)])

    return sc_scatter


def kernel(h, e, edge_index, WA, bA, WB, bB, WC, bC, WD, bD, WE, bE, P):
    N, D = h.shape
    E = e.shape[0]
    K = 5
    Es = E // K
    R2 = 512

    src = edge_index[0]
    dst = edge_index[1]
    dstn = dst + N
    p = jnp.clip(P, 1.0, 100.0).reshape(1, D)
    zeros = jnp.zeros((N, D), jnp.float32)

    de, bhp = _node_proj(h, WB, bB.reshape(1, D), WD, bD.reshape(1, D),
                         WE, bE.reshape(1, D), p, R=1000)
    table = de.reshape(2 * N, D)

    enew_buf = None
    sums = []
    for k in range(K):
        g1_k = _make_sc_gather(2 * N, Es, D, 200, k * Es)(table, src, dstn)
        enew_buf, sig_k = _edge_stage(e, g1_k, WC, bC.reshape(1, D), p,
                                      R2, k * (Es // R2), enew_buf)
        sums.append(_make_sc_scatter(N, Es, D, 80, k * Es)(
            sig_k, bhp, src, dst, zeros))

    h_new = _node_out(h, WA, bA.reshape(1, D),
                      [sm for sm, _ in sums], [ss for _, ss in sums],
                      p, R=1000)

    return (h_new, enew_buf)


# speedup vs baseline: 4.9771x; 1.0021x over previous
import functools

import jax
import jax.numpy as jnp
from jax import lax
from jax.experimental import pallas as pl
from jax.experimental.pallas import tpu as pltpu
from jax.experimental.pallas import tpu_sc as plsc

NC = 2
NS = 16


def _node_proj_body(h_ref, WB_ref, bB_ref, WD_ref, bD_ref, WE_ref, bE_ref,
                    p_ref, de_ref, bhp_ref):
    hb = h_ref[...]
    dn = (((1,), (1,)), ((), ()))
    Dh = lax.dot_general(hb, WD_ref[...], dn,
                         preferred_element_type=jnp.float32) + bD_ref[...]
    Eh = lax.dot_general(hb, WE_ref[...], dn,
                         preferred_element_type=jnp.float32) + bE_ref[...]
    Bh = lax.dot_general(hb, WB_ref[...], dn,
                         preferred_element_type=jnp.float32) + bB_ref[...]
    de_ref[0] = Dh
    de_ref[1] = Eh
    bhp_ref[...] = jnp.exp(p_ref[...] * jnp.log(jnp.abs(Bh)))


def _node_proj(h, WB, bB, WD, bD, WE, bE, p, R):
    N, D = h.shape
    grid = (N // R,)
    full = pl.BlockSpec((D, D), lambda i: (0, 0))
    vec = pl.BlockSpec((1, D), lambda i: (0, 0))
    return pl.pallas_call(
        _node_proj_body,
        grid=grid,
        in_specs=[
            pl.BlockSpec((R, D), lambda i: (i, 0)),
            full, vec, full, vec, full, vec, vec,
        ],
        out_specs=[
            pl.BlockSpec((2, R, D), lambda i: (0, i, 0)),
            pl.BlockSpec((R, D), lambda i: (i, 0)),
        ],
        out_shape=[
            jax.ShapeDtypeStruct((2, N, D), jnp.float32),
            jax.ShapeDtypeStruct((N, D), jnp.float32),
        ],
    )(h, WB, bB, WD, bD, WE, bE, p)


def _edge_body(e_ref, g1_ref, WC_ref, bC_ref, p_ref, enew_ref, sig_ref):
    eb = e_ref[...]
    dn = (((1,), (1,)), ((), ()))
    Ce = lax.dot_general(eb, WC_ref[...], dn,
                         preferred_element_type=jnp.float32) + bC_ref[...]
    e_out = Ce + g1_ref[...]
    sigma = jax.nn.sigmoid(e_out)
    sig_ref[...] = jnp.exp(p_ref[...] * jnp.log(sigma))
    enew_ref[...] = eb + jnp.maximum(e_out, 0.0)


def _edge_body_alias(enew_in_ref, e_ref, g1_ref, WC_ref, bC_ref, p_ref,
                     enew_ref, sig_ref):
    _edge_body(e_ref, g1_ref, WC_ref, bC_ref, p_ref, enew_ref, sig_ref)


def _edge_stage(e, g1_k, WC, bC, p, R, koff, enew_in):
    E, D = e.shape
    Es = g1_k.shape[0]
    grid = (Es // R,)
    full = pl.BlockSpec((D, D), lambda i: (0, 0))
    vec = pl.BlockSpec((1, D), lambda i: (0, 0))
    blk = pl.BlockSpec((R, D), lambda i: (i, 0))
    eblk = pl.BlockSpec((R, D), lambda i: (koff + i, 0))
    out_shape = [
        jax.ShapeDtypeStruct((E, D), jnp.float32),
        jax.ShapeDtypeStruct((Es, D), jnp.float32),
    ]
    if enew_in is None:
        return pl.pallas_call(
            _edge_body,
            grid=grid,
            in_specs=[eblk, blk, full, vec, vec],
            out_specs=[eblk, blk],
            out_shape=out_shape,
        )(e, g1_k, WC, bC, p)
    return pl.pallas_call(
        _edge_body_alias,
        grid=grid,
        in_specs=[pl.BlockSpec(memory_space=pltpu.HBM),
                  eblk, blk, full, vec, vec],
        out_specs=[eblk, blk],
        out_shape=out_shape,
        input_output_aliases={0: 0},
    )(enew_in, e, g1_k, WC, bC, p)


def _make_node_out_body(K):
    def body(*refs):
        h_ref, WA_ref, bA_ref = refs[0], refs[1], refs[2]
        sm_refs = refs[3:3 + K]
        ss_refs = refs[3 + K:3 + 2 * K]
        p_ref = refs[3 + 2 * K]
        out_ref = refs[4 + 2 * K]
        hb = h_ref[...]
        dn = (((1,), (1,)), ((), ()))
        Ah = lax.dot_general(hb, WA_ref[...], dn,
                             preferred_element_type=jnp.float32) + bA_ref[...]
        sm = sm_refs[0][...]
        for r in sm_refs[1:]:
            sm = sm + r[...]
        ss = ss_refs[0][...]
        for r in ss_refs[1:]:
            ss = ss + r[...]
        ratio = sm / (ss + 1e-6)
        root = jnp.exp(jnp.log(ratio) / p_ref[...])
        out_ref[...] = hb + jnp.maximum(Ah + root, 0.0)
    return body


def _node_out(h, WA, bA, sum_ms, sum_ss, p, R):
    N, D = h.shape
    K = len(sum_ms)
    grid = (N // R,)
    full = pl.BlockSpec((D, D), lambda i: (0, 0))
    vec = pl.BlockSpec((1, D), lambda i: (0, 0))
    blk = pl.BlockSpec((R, D), lambda i: (i, 0))
    return pl.pallas_call(
        _make_node_out_body(K),
        grid=grid,
        in_specs=[blk, full, vec] + [blk] * (2 * K) + [vec],
        out_specs=blk,
        out_shape=jax.ShapeDtypeStruct((N, D), jnp.float32),
    )(h, WA, bA, *sum_ms, *sum_ss, p)


def _make_sc_gather(N2, E, D, B, goff):
    EPW = E // (NC * NS)
    NCHUNK = EPW // B
    T = NCHUNK // 2
    mesh = plsc.VectorSubcoreMesh(core_axis_name="c", subcore_axis_name="s")

    @functools.partial(
        pl.kernel,
        out_type=jax.ShapeDtypeStruct((E, D), jnp.float32),
        mesh=mesh,
        scratch_types=[
            pltpu.VMEM((B,), jnp.int32),
            pltpu.VMEM((B,), jnp.int32),
            pltpu.VMEM((B,), jnp.int32),
            pltpu.VMEM((B,), jnp.int32),
            pltpu.VMEM((B, D), jnp.float32),
            pltpu.VMEM((B, D), jnp.float32),
            pltpu.VMEM((B, D), jnp.float32),
            pltpu.VMEM((B, D), jnp.float32),
        ] + [pltpu.SemaphoreType.DMA] * 10,
    )
    def sc_gather(table_hbm, src_hbm, dstn_hbm, out_hbm,
                  sv0, sv1, dv0, dv1, r1a, r1b, r2a, r2b,
                  ssi0, ssi1, sdi0, sdi1, sg10, sg11, sg20, sg21, wr0, wr1):
        c = lax.axis_index("c")
        s = lax.axis_index("s")
        wid = s * NC + c
        w0 = goff + wid * EPW
        o0 = wid * EPW
        srcv = (sv0, sv1)
        dstv = (dv0, dv1)
        r1 = (r1a, r1b)
        r2 = (r2a, r2b)
        ssi = (ssi0, ssi1)
        sdi = (sdi0, sdi1)
        sg1 = (sg10, sg11)
        sg2 = (sg20, sg21)
        wr = (wr0, wr1)

        def add_rows(ra, rb):
            def row(r, carry):
                for j in range(D // 16):
                    sl = pl.ds(j * 16, 16)
                    ra[r, sl] = ra[r, sl] + rb[r, sl]
                return carry
            lax.fori_loop(0, B, row, 0)

        pltpu.sync_copy(src_hbm.at[pl.ds(w0, B)], srcv[0])
        pltpu.sync_copy(dstn_hbm.at[pl.ds(w0, B)], dstv[0])
        pltpu.async_copy(src_hbm.at[pl.ds(w0 + B, B)], srcv[1], ssi[1])
        pltpu.async_copy(dstn_hbm.at[pl.ds(w0 + B, B)], dstv[1], sdi[1])
        pltpu.async_copy(table_hbm.at[srcv[0]], r1[0], sg1[0])
        pltpu.async_copy(table_hbm.at[dstv[0]], r2[0], sg2[0])

        def outer(t, carry):
            for k in (0, 1):
                pi = k
                po = 1 - k
                g = 2 * t + k
                base = w0 + g * B
                obase = o0 + g * B

                if k == 0:
                    @pl.when(t > 0)
                    def _():
                        pltpu.make_async_copy(
                            r1[po], out_hbm.at[pl.ds(obase - B, B)],
                            wr[po]).wait()
                else:
                    wrd[0].wait()

                if k == 0:
                    pltpu.make_async_copy(
                        table_hbm.at[srcv[pi]], r1[pi], sg1[pi]).wait()
                    pltpu.make_async_copy(
                        table_hbm.at[dstv[pi]], r2[pi], sg2[pi]).wait()
                else:
                    gd[0].wait()
                    gd[1].wait()

                @pl.when(g + 2 < NCHUNK)
                def _():
                    pltpu.async_copy(src_hbm.at[pl.ds(base + 2 * B, B)],
                                     srcv[pi], ssi[pi])
                    pltpu.async_copy(dstn_hbm.at[pl.ds(base + 2 * B, B)],
                                     dstv[pi], sdi[pi])

                @pl.when(g + 1 < NCHUNK)
                def _():
                    pltpu.make_async_copy(
                        src_hbm.at[pl.ds(base + B, B)], srcv[po],
                        ssi[po]).wait()
                    pltpu.make_async_copy(
                        dstn_hbm.at[pl.ds(base + B, B)], dstv[po],
                        sdi[po]).wait()
                    d1 = pltpu.async_copy(table_hbm.at[srcv[po]],
                                          r1[po], sg1[po])
                    d2 = pltpu.async_copy(table_hbm.at[dstv[po]],
                                          r2[po], sg2[po])

                if k == 0:
                    gd = (pltpu.make_async_copy(table_hbm.at[srcv[po]],
                                                r1[po], sg1[po]),
                          pltpu.make_async_copy(table_hbm.at[dstv[po]],
                                                r2[po], sg2[po]))

                add_rows(r1[pi], r2[pi])

                wd = pltpu.async_copy(r1[pi], out_hbm.at[pl.ds(obase, B)],
                                      wr[pi])
                if k == 0:
                    wrd = (wd,)
            return carry

        lax.fori_loop(0, T, outer, 0)
        pltpu.make_async_copy(
            r1[1], out_hbm.at[pl.ds(o0 + (NCHUNK - 1) * B, B)], wr[1]).wait()

    return sc_gather


def _make_sc_scatter(N, E, D, B, goff, mrole):
    EPS = E // NS
    NCHUNK = EPS // B
    RPS = (N // NS) // 8 * 8
    TAIL = N - NS * RPS
    mesh = plsc.VectorSubcoreMesh(core_axis_name="c", subcore_axis_name="s")

    T = NCHUNK // 2

    @functools.partial(
        pl.kernel,
        out_type=[
            jax.ShapeDtypeStruct((N, D), jnp.float32),
            jax.ShapeDtypeStruct((N, D), jnp.float32),
        ],
        mesh=mesh,
        scratch_types=[
            pltpu.VMEM((B,), jnp.int32),
            pltpu.VMEM((B,), jnp.int32),
            pltpu.VMEM((B,), jnp.int32),
            pltpu.VMEM((B,), jnp.int32),
            pltpu.VMEM((B, D), jnp.float32),
            pltpu.VMEM((B, D), jnp.float32),
            pltpu.VMEM((B, D), jnp.float32),
            pltpu.VMEM((B, D), jnp.float32),
            pltpu.VMEM_SHARED((N, D), jnp.float32),
        ] + [pltpu.SemaphoreType.DMA] * 10,
    )
    def sc_scatter(sig_hbm, bhp_hbm, src_hbm, dst_hbm, zero_hbm,
                   out_m, out_s, sv0, sv1, dv0, dv1, sa, sb, ba, bb, acc,
                   ssi0, ssi1, sdi0, sdi1, sg0, sg1, ss0, ss1, sc0, sc1):
        c = lax.axis_index("c")
        s = lax.axis_index("s")
        w0 = goff + s * EPS
        l0 = s * EPS
        srcv = (sv0, sv1)
        dstv = (dv0, dv1)
        rs = (sa, sb)
        rb = (ba, bb)
        ssi = (ssi0, ssi1)
        sdi = (sdi0, sdi1)
        sg = (sg0, sg1)
        ss = (ss0, ss1)
        sc = (sc0, sc1)

        pltpu.sync_copy(zero_hbm.at[pl.ds(s * RPS, RPS)],
                        acc.at[pl.ds(s * RPS, RPS)])

        @pl.when(s == 0)
        def _():
            pltpu.sync_copy(zero_hbm.at[pl.ds(NS * RPS, TAIL)],
                            acc.at[pl.ds(NS * RPS, TAIL)])

        plsc.subcore_barrier()

        def mul_rows(ra, rbb):
            def row(r, carry):
                for j in range(D // 16):
                    sl = pl.ds(j * 16, 16)
                    ra[r, sl] = ra[r, sl] * rbb[r, sl]
                return carry
            lax.fori_loop(0, B, row, 0)

        @pl.when(c == mrole)
        def _():
            pltpu.sync_copy(src_hbm.at[pl.ds(w0, B)], srcv[0])
            pltpu.sync_copy(dst_hbm.at[pl.ds(w0, B)], dstv[0])
            pltpu.async_copy(src_hbm.at[pl.ds(w0 + B, B)], srcv[1], ssi[1])
            pltpu.async_copy(bhp_hbm.at[srcv[0]], rb[0], sg[0])
            pltpu.async_copy(sig_hbm.at[pl.ds(l0, B)], rs[0], ss[0])

            def outer(t, carry):
                for k in (0, 1):
                    pi = k
                    po = 1 - k
                    g = 2 * t + k
                    base = w0 + g * B
                    lbase = l0 + g * B

                    if k == 0:
                        @pl.when(t > 0)
                        def _():
                            pltpu.make_async_copy(
                                rs[po], acc.at[dstv[po]], sc[po]).wait()
                    else:
                        scd[0].wait()

                    @pl.when(g + 1 < NCHUNK)
                    def _():
                        pltpu.async_copy(dst_hbm.at[pl.ds(base + B, B)],
                                         dstv[po], sdi[po])

                    if k == 0:
                        pltpu.make_async_copy(
                            bhp_hbm.at[srcv[pi]], rb[pi], sg[pi]).wait()
                        pltpu.make_async_copy(
                            sig_hbm.at[pl.ds(lbase, B)], rs[pi],
                            ss[pi]).wait()
                    else:
                        gd[0].wait()
                        gd[1].wait()

                    @pl.when(g + 2 < NCHUNK)
                    def _():
                        pltpu.async_copy(src_hbm.at[pl.ds(base + 2 * B, B)],
                                         srcv[pi], ssi[pi])

                    @pl.when(g + 1 < NCHUNK)
                    def _():
                        pltpu.make_async_copy(
                            src_hbm.at[pl.ds(base + B, B)], srcv[po],
                            ssi[po]).wait()
                        pltpu.make_async_copy(
                            dst_hbm.at[pl.ds(base + B, B)], dstv[po],
                            sdi[po]).wait()
                        pltpu.async_copy(bhp_hbm.at[srcv[po]],
                                         rb[po], sg[po])
                        pltpu.async_copy(sig_hbm.at[pl.ds(lbase + B, B)],
                                         rs[po], ss[po])

                    if k == 0:
                        gd = (pltpu.make_async_copy(bhp_hbm.at[srcv[po]],
                                                    rb[po], sg[po]),
                              pltpu.make_async_copy(
                                  sig_hbm.at[pl.ds(lbase + B, B)], rs[po],
                                  ss[po]))

                    mul_rows(rs[pi], rb[pi])

                    sd = pltpu.async_copy(rs[pi], acc.at[dstv[pi]],
                                          sc[pi], add=True)
                    if k == 0:
                        scd = (sd,)
                return carry

            lax.fori_loop(0, T, outer, 0)
            pltpu.make_async_copy(rs[1], acc.at[dstv[1]], sc[1]).wait()

        @pl.when(c == 1 - mrole)
        def _():
            pltpu.async_copy(sig_hbm.at[pl.ds(l0, B)], rs[0], ss[0])
            pltpu.async_copy(dst_hbm.at[pl.ds(w0, B)], dstv[0], sdi[0])

            def outer(t, carry):
                for k in (0, 1):
                    pi = k
                    po = 1 - k
                    g = 2 * t + k
                    base = w0 + g * B
                    lbase = l0 + g * B

                    if k == 0:
                        @pl.when(t > 0)
                        def _():
                            pltpu.make_async_copy(
                                rs[po], acc.at[dstv[po]], sc[po]).wait()
                    else:
                        scd[0].wait()

                    @pl.when(g + 1 < NCHUNK)
                    def _():
                        pltpu.async_copy(sig_hbm.at[pl.ds(lbase + B, B)],
                                         rs[po], ss[po])
                        pltpu.async_copy(dst_hbm.at[pl.ds(base + B, B)],
                                         dstv[po], sdi[po])

                    pltpu.make_async_copy(
                        sig_hbm.at[pl.ds(lbase, B)], rs[pi], ss[pi]).wait()
                    pltpu.make_async_copy(
                        dst_hbm.at[pl.ds(base, B)], dstv[pi],
                        sdi[pi]).wait()

                    sd = pltpu.async_copy(rs[pi], acc.at[dstv[pi]],
                                          sc[pi], add=True)
                    if k == 0:
                        scd = (sd,)
                return carry

            lax.fori_loop(0, T, outer, 0)
            pltpu.make_async_copy(rs[1], acc.at[dstv[1]], sc[1]).wait()

        plsc.subcore_barrier()

        @pl.when(c == mrole)
        def _():
            pltpu.sync_copy(acc.at[pl.ds(s * RPS, RPS)],
                            out_m.at[pl.ds(s * RPS, RPS)])

            @pl.when(s == 0)
            def _():
                pltpu.sync_copy(acc.at[pl.ds(NS * RPS, TAIL)],
                                out_m.at[pl.ds(NS * RPS, TAIL)])

        @pl.when(c == 1 - mrole)
        def _():
            pltpu.sync_copy(acc.at[pl.ds(s * RPS, RPS)],
                            out_s.at[pl.ds(s * RPS, RPS)])

            @pl.when(s == 0)
            def _():
                pltpu.sync_copy(acc.at[pl.ds(NS * RPS, TAIL)],
                                out_s.at[pl.ds(NS * RPS, TAIL)])

    return sc_scatter


def kernel(h, e, edge_index, WA, bA, WB, bB, WC, bC, WD, bD, WE, bE, P):
    N, D = h.shape
    E = e.shape[0]
    K = 5
    Es = E // K
    R2 = 512

    src = edge_index[0]
    dst = edge_index[1]
    dstn = dst + N
    p = jnp.clip(P, 1.0, 100.0).reshape(1, D)
    zeros = jnp.zeros((N, D), jnp.float32)

    de, bhp = _node_proj(h, WB, bB.reshape(1, D), WD, bD.reshape(1, D),
                         WE, bE.reshape(1, D), p, R=1000)
    table = de.reshape(2 * N, D)

    enew_buf = None
    sums = []
    for k in range(K):
        g1_k = _make_sc_gather(2 * N, Es, D, 200, k * Es)(table, src, dstn)
        enew_buf, sig_k = _edge_stage(e, g1_k, WC, bC.reshape(1, D), p,
                                      R2, k * (Es // R2), enew_buf)
        sums.append(_make_sc_scatter(N, Es, D, 80, k * Es, k % 2)(
            sig_k, bhp, src, dst, zeros))

    h_new = _node_out(h, WA, bA.reshape(1, D),
                      [sm for sm, _ in sums], [ss for _, ss in sums],
                      p, R=1000)

    return (h_new, enew_buf)
